# baseline (speedup 1.0000x reference)
import jax
import jax.numpy as jnp
from jax import lax
from jax.experimental import pallas as pl
from jax.experimental.pallas import tpu as pltpu

ROWS = 128
COLS = 32768
BR = 16
NBLK = ROWS // BR
NBUF = 8
_BIG = 1e9


def _blk_argmax(xb):
    m = jnp.max(xb, axis=1, keepdims=True)
    iota = lax.broadcasted_iota(jnp.int32, (BR, COLS), 1).astype(jnp.float32)
    masked = jnp.where(xb == m, iota, jnp.full((), _BIG, jnp.float32))
    return jnp.min(masked, axis=1).astype(jnp.int32)


def _body(x_hbm, o_ref, buf, sems):
    def copy(b):
        return pltpu.make_async_copy(
            x_hbm.at[pl.ds(b * BR, BR), :], buf.at[b % NBUF], sems.at[b % NBUF])

    for b in range(NBUF):
        copy(b).start()
    for b in range(NBLK):
        copy(b).wait()
        o_ref[b, 0, :] = _blk_argmax(buf[b % NBUF])
        if b + NBUF < NBLK:
            copy(b + NBUF).start()


@jax.jit
def kernel(x):
    out = pl.pallas_call(
        _body,
        in_specs=[pl.BlockSpec(memory_space=pltpu.MemorySpace.HBM)],
        out_specs=pl.BlockSpec(memory_space=pltpu.MemorySpace.VMEM),
        out_shape=jax.ShapeDtypeStruct((NBLK, 1, BR), jnp.int32),
        scratch_shapes=[
            pltpu.VMEM((NBUF, BR, COLS), jnp.float32),
            pltpu.SemaphoreType.DMA((NBUF,)),
        ],
    )(x)
    return out.reshape(ROWS).astype(jnp.int64)


# speedup vs baseline: 1.2999x; 1.2999x over previous
import jax
import jax.numpy as jnp
from jax import lax
from jax.experimental import pallas as pl
from jax.experimental.pallas import tpu as pltpu

ROWS = 128
COLS = 32768
BR = 16
NBLK = ROWS // BR
NBUF = 8
_BIG = 1e9


def _blk_argmax(xb):
    m = jnp.max(xb, axis=1, keepdims=True)
    iota = lax.broadcasted_iota(jnp.int32, (BR, COLS), 1).astype(jnp.float32)
    masked = jnp.where(xb == m, iota, jnp.full((), _BIG, jnp.float32))
    return jnp.min(masked, axis=1).astype(jnp.int32)


def _body(x_hbm, o_ref, buf, sems):
    def copy(b):
        return pltpu.make_async_copy(
            x_hbm.at[pl.ds(b * BR, BR), :], buf.at[b % NBUF], sems.at[b % NBUF])

    for b in range(NBUF):
        copy(b).start()
    for b in range(NBLK):
        copy(b).wait()
        o_ref[0, pl.ds(b * BR, BR)] = _blk_argmax(buf[b % NBUF])
        if b + NBUF < NBLK:
            copy(b + NBUF).start()


@jax.jit
def kernel(x):
    out = pl.pallas_call(
        _body,
        in_specs=[pl.BlockSpec(memory_space=pltpu.MemorySpace.HBM)],
        out_specs=pl.BlockSpec(memory_space=pltpu.MemorySpace.VMEM),
        out_shape=jax.ShapeDtypeStruct((1, ROWS), jnp.int32),
        scratch_shapes=[
            pltpu.VMEM((NBUF, BR, COLS), jnp.float32),
            pltpu.SemaphoreType.DMA((NBUF,)),
        ],
    )(x)
    return out.reshape(ROWS).astype(jnp.int64)
